# baseline (speedup 1.0000x reference)
import jax
import jax.numpy as jnp
from jax.experimental import pallas as pl
from jax.experimental.pallas import tpu as pltpu

ROWS = 4096
COLS = 16384
R_BLK = 128
C_BLK = 4096
CHUNK = 128


def _cumsum_kernel(x_ref, t3_ref, out_ref, carry_ref):
    ct = pl.program_id(1)

    @pl.when(ct == 0)
    def _init():
        carry_ref[...] = jnp.zeros_like(carry_ref)

    carry = carry_ref[...]
    t3 = t3_ref[...]
    for c in range(C_BLK // CHUNK):
        xc = x_ref[:, c * CHUNK:(c + 1) * CHUNK]
        hi = xc.astype(jnp.bfloat16)
        lo = (xc - hi.astype(jnp.float32)).astype(jnp.bfloat16)
        hl = jnp.concatenate([hi, lo], axis=1)
        res = jnp.dot(hl, t3, preferred_element_type=jnp.float32)
        out_ref[:, c * CHUNK:(c + 1) * CHUNK] = res[:, :CHUNK] + carry
        carry = carry + res[:, CHUNK:]
    carry_ref[...] = carry


@jax.jit
def kernel(x):
    tri = jnp.triu(jnp.ones((CHUNK, CHUNK), dtype=jnp.bfloat16))
    t2 = jnp.concatenate(
        [tri, jnp.ones((CHUNK, CHUNK), dtype=jnp.bfloat16)], axis=1)
    t3 = jnp.concatenate([t2, t2], axis=0)
    grid = (ROWS // R_BLK, COLS // C_BLK)
    return pl.pallas_call(
        _cumsum_kernel,
        grid=grid,
        in_specs=[
            pl.BlockSpec((R_BLK, C_BLK), lambda i, j: (i, j)),
            pl.BlockSpec((2 * CHUNK, 2 * CHUNK), lambda i, j: (0, 0)),
        ],
        out_specs=pl.BlockSpec((R_BLK, C_BLK), lambda i, j: (i, j)),
        out_shape=jax.ShapeDtypeStruct((ROWS, COLS), jnp.float32),
        scratch_shapes=[pltpu.VMEM((R_BLK, CHUNK), jnp.float32)],
        compiler_params=pltpu.CompilerParams(
            dimension_semantics=("arbitrary", "arbitrary"),
        ),
    )(x, t3)


# speedup vs baseline: 1.2339x; 1.2339x over previous
import jax
import jax.numpy as jnp
from jax.experimental import pallas as pl
from jax.experimental.pallas import tpu as pltpu

ROWS = 4096
COLS = 16384
R_BLK = 512
C_BLK = 2048
CHUNK = 128


def _cumsum_kernel(x_ref, t3_ref, out_ref, carry_ref):
    ct = pl.program_id(1)

    @pl.when(ct == 0)
    def _init():
        carry_ref[...] = jnp.zeros_like(carry_ref)

    carry = carry_ref[...]
    t3 = t3_ref[...]
    for c in range(C_BLK // CHUNK):
        xc = x_ref[:, c * CHUNK:(c + 1) * CHUNK]
        hi = xc.astype(jnp.bfloat16)
        lo = (xc - hi.astype(jnp.float32)).astype(jnp.bfloat16)
        hl = jnp.concatenate([hi, lo], axis=1)
        res = jnp.dot(hl, t3, preferred_element_type=jnp.float32)
        out_ref[:, c * CHUNK:(c + 1) * CHUNK] = res[:, :CHUNK] + carry
        carry = carry + res[:, CHUNK:]
    carry_ref[...] = carry


@jax.jit
def kernel(x):
    tri = jnp.triu(jnp.ones((CHUNK, CHUNK), dtype=jnp.bfloat16))
    t2 = jnp.concatenate(
        [tri, jnp.ones((CHUNK, CHUNK), dtype=jnp.bfloat16)], axis=1)
    t3 = jnp.concatenate([t2, t2], axis=0)
    grid = (ROWS // R_BLK, COLS // C_BLK)
    return pl.pallas_call(
        _cumsum_kernel,
        grid=grid,
        in_specs=[
            pl.BlockSpec((R_BLK, C_BLK), lambda i, j: (i, j)),
            pl.BlockSpec((2 * CHUNK, 2 * CHUNK), lambda i, j: (0, 0)),
        ],
        out_specs=pl.BlockSpec((R_BLK, C_BLK), lambda i, j: (i, j)),
        out_shape=jax.ShapeDtypeStruct((ROWS, COLS), jnp.float32),
        scratch_shapes=[pltpu.VMEM((R_BLK, CHUNK), jnp.float32)],
        compiler_params=pltpu.CompilerParams(
            dimension_semantics=("arbitrary", "arbitrary"),
        ),
    )(x, t3)


# speedup vs baseline: 1.2787x; 1.0363x over previous
import jax
import jax.numpy as jnp
from jax.experimental import pallas as pl
from jax.experimental.pallas import tpu as pltpu

ROWS = 4096
COLS = 16384
R_BLK = 512
C_BLK = 4096
CHUNK = 128


def _cumsum_kernel(x_ref, t3_ref, out_ref, carry_ref):
    ct = pl.program_id(1)

    @pl.when(ct == 0)
    def _init():
        carry_ref[...] = jnp.zeros_like(carry_ref)

    carry = carry_ref[...]
    t3 = t3_ref[...]
    for c in range(C_BLK // CHUNK):
        xc = x_ref[:, c * CHUNK:(c + 1) * CHUNK]
        hi = xc.astype(jnp.bfloat16)
        lo = (xc - hi.astype(jnp.float32)).astype(jnp.bfloat16)
        hl = jnp.concatenate([hi, lo], axis=1)
        res = jnp.dot(hl, t3, preferred_element_type=jnp.float32)
        out_ref[:, c * CHUNK:(c + 1) * CHUNK] = res[:, :CHUNK] + carry
        carry = carry + res[:, CHUNK:]
    carry_ref[...] = carry


@jax.jit
def kernel(x):
    tri = jnp.triu(jnp.ones((CHUNK, CHUNK), dtype=jnp.bfloat16))
    t2 = jnp.concatenate(
        [tri, jnp.ones((CHUNK, CHUNK), dtype=jnp.bfloat16)], axis=1)
    t3 = jnp.concatenate([t2, t2], axis=0)
    grid = (ROWS // R_BLK, COLS // C_BLK)
    return pl.pallas_call(
        _cumsum_kernel,
        grid=grid,
        in_specs=[
            pl.BlockSpec((R_BLK, C_BLK), lambda i, j: (i, j)),
            pl.BlockSpec((2 * CHUNK, 2 * CHUNK), lambda i, j: (0, 0)),
        ],
        out_specs=pl.BlockSpec((R_BLK, C_BLK), lambda i, j: (i, j)),
        out_shape=jax.ShapeDtypeStruct((ROWS, COLS), jnp.float32),
        scratch_shapes=[pltpu.VMEM((R_BLK, CHUNK), jnp.float32)],
        compiler_params=pltpu.CompilerParams(
            dimension_semantics=("arbitrary", "arbitrary"),
        ),
    )(x, t3)


# speedup vs baseline: 1.2793x; 1.0004x over previous
import jax
import jax.numpy as jnp
from jax.experimental import pallas as pl
from jax.experimental.pallas import tpu as pltpu

ROWS = 4096
COLS = 16384
R_BLK = 256
C_BLK = 8192
CHUNK = 128


def _cumsum_kernel(x_ref, t3_ref, out_ref, carry_ref):
    ct = pl.program_id(1)

    @pl.when(ct == 0)
    def _init():
        carry_ref[...] = jnp.zeros_like(carry_ref)

    carry = carry_ref[...]
    t3 = t3_ref[...]
    for c in range(C_BLK // CHUNK):
        xc = x_ref[:, c * CHUNK:(c + 1) * CHUNK]
        hi = xc.astype(jnp.bfloat16)
        lo = (xc - hi.astype(jnp.float32)).astype(jnp.bfloat16)
        hl = jnp.concatenate([hi, lo], axis=1)
        res = jnp.dot(hl, t3, preferred_element_type=jnp.float32)
        out_ref[:, c * CHUNK:(c + 1) * CHUNK] = res[:, :CHUNK] + carry
        carry = carry + res[:, CHUNK:]
    carry_ref[...] = carry


@jax.jit
def kernel(x):
    tri = jnp.triu(jnp.ones((CHUNK, CHUNK), dtype=jnp.bfloat16))
    t2 = jnp.concatenate(
        [tri, jnp.ones((CHUNK, CHUNK), dtype=jnp.bfloat16)], axis=1)
    t3 = jnp.concatenate([t2, t2], axis=0)
    grid = (ROWS // R_BLK, COLS // C_BLK)
    return pl.pallas_call(
        _cumsum_kernel,
        grid=grid,
        in_specs=[
            pl.BlockSpec((R_BLK, C_BLK), lambda i, j: (i, j)),
            pl.BlockSpec((2 * CHUNK, 2 * CHUNK), lambda i, j: (0, 0)),
        ],
        out_specs=pl.BlockSpec((R_BLK, C_BLK), lambda i, j: (i, j)),
        out_shape=jax.ShapeDtypeStruct((ROWS, COLS), jnp.float32),
        scratch_shapes=[pltpu.VMEM((R_BLK, CHUNK), jnp.float32)],
        compiler_params=pltpu.CompilerParams(
            dimension_semantics=("arbitrary", "arbitrary"),
        ),
    )(x, t3)
